# baseline (speedup 1.0000x reference)
import functools

import jax
import jax.numpy as jnp
from jax import lax
from jax.experimental import pallas as pl
from jax.experimental.pallas import tpu as pltpu
from jax.experimental.pallas import tpu_sc as plsc

NC = 2
NS = 16
CH = 128
W16 = 16


def _sc_mesh():
    return plsc.VectorSubcoreMesh(core_axis_name="c", subcore_axis_name="s")


def _deg_partials(dst_w, T, AR, NCH, D):
    @functools.partial(
        pl.kernel,
        out_type=jax.ShapeDtypeStruct((NC, T, D), jnp.float32),
        mesh=_sc_mesh(),
        scratch_types=[
            pltpu.VMEM((NCH, CH), jnp.int32),
            pltpu.VMEM((CH, D), jnp.float32),
            pltpu.VMEM_SHARED((T, D), jnp.float32),
            pltpu.SemaphoreType.DMA,
        ],
    )
    def deg_k(dstw_h, degp_h, idx_v, buf_v, dacc, sem):
        cid = lax.axis_index("c")
        sid = lax.axis_index("s")
        w = cid * NS + sid
        pltpu.sync_copy(dstw_h.at[w], idx_v)
        for i in range(CH):
            for c in range(D // 16):
                buf_v[i, pl.ds(c * 16, 16)] = jnp.zeros((16,), jnp.float32)
        base = sid * AR
        for c in range(AR // CH):
            pltpu.sync_copy(buf_v, dacc.at[pl.ds(base + c * CH, CH)])
        for i in range(CH):
            for c in range(D // 16):
                buf_v[i, pl.ds(c * 16, 16)] = jnp.ones((16,), jnp.float32)
        plsc.subcore_barrier()
        descs = [
            pltpu.async_copy(buf_v, dacc.at[idx_v.at[j]], sem, add=True)
            for j in range(NCH)
        ]
        for dsc in descs:
            dsc.wait()
        plsc.subcore_barrier()
        for c in range(AR // CH):
            pltpu.sync_copy(dacc.at[pl.ds(base + c * CH, CH)], buf_v)
            pltpu.sync_copy(buf_v, degp_h.at[cid, pl.ds(base + c * CH, CH)])

    return deg_k(dst_w)


AB = 2
ACH = 80


def _agg_partials(xs, idx_tab, T, AR, NG0, NG1, D):
    @functools.partial(
        pl.kernel,
        out_type=jax.ShapeDtypeStruct((NC, T, D), jnp.float32),
        mesh=_sc_mesh(),
        scratch_types=[
            pltpu.VMEM((2, 2 * AB, ACH), jnp.int32),
            pltpu.VMEM((AB, ACH, D), jnp.float32),
            pltpu.VMEM_SHARED((T, D), jnp.float32),
        ] + [pltpu.SemaphoreType.DMA] * (2 + 2 * AB),
    )
    def agg_k(xs_h, idx_h, aggp_h, idxb, rows, acc, *sems):
        isem, gsem, ssem = sems[:2], sems[2:2 + AB], sems[2 + AB:]
        cid = lax.axis_index("c")
        sid = lax.axis_index("s")
        w = cid * NS + sid
        for i in range(ACH):
            for c in range(D // 16):
                rows[0, i, pl.ds(c * 16, 16)] = jnp.zeros((16,), jnp.float32)
        base = sid * AR
        for c in range(AR // ACH):
            pltpu.sync_copy(rows.at[0], acc.at[pl.ds(base + c * ACH, ACH)])
        plsc.subcore_barrier()

        def run(NG):
            pltpu.sync_copy(idx_h.at[w, 0], idxb.at[0])
            idesc = [None, None]
            if NG > 1:
                idesc[1] = pltpu.async_copy(idx_h.at[w, 1], idxb.at[1], isem[1])
            gd = [
                pltpu.async_copy(xs_h.at[idxb.at[0, b]], rows.at[b], gsem[b])
                for b in range(AB)
            ]
            sd = [None] * AB
            for g in range(NG):
                p = g % 2
                for b in range(AB):
                    gd[b].wait()
                    sd[b] = pltpu.async_copy(
                        rows.at[b], acc.at[idxb.at[p, AB + b]], ssem[b],
                        add=True)
                if g + 1 < NG:
                    idesc[1 - p].wait()
                    for b in range(AB):
                        sd[b].wait()
                        gd[b] = pltpu.async_copy(
                            xs_h.at[idxb.at[1 - p, b]], rows.at[b], gsem[b])
                    if g + 2 < NG:
                        idesc[p] = pltpu.async_copy(
                            idx_h.at[w, g + 2], idxb.at[p], isem[p])
                else:
                    for b in range(AB):
                        sd[b].wait()

        @pl.when(cid == 0)
        def _():
            run(NG0)

        @pl.when(cid == 1)
        def _():
            run(NG1)

        plsc.subcore_barrier()
        for c in range(AR // ACH):
            pltpu.sync_copy(acc.at[pl.ds(base + c * ACH, ACH)], rows.at[0])
            pltpu.sync_copy(
                rows.at[0], aggp_h.at[cid, pl.ds(base + c * ACH, ACH)])

    return agg_k(xs, idx_tab)


def _tc_prep(x, W0, degp, N, T, D):
    def body(x_ref, w_ref, dp_ref, xs_ref, dinv_ref):
        deg = dp_ref[0, :, 0:1] + dp_ref[1, :, 0:1] + 1.0
        dinv = lax.rsqrt(deg)
        dinv_ref[...] = dinv
        xw = jnp.dot(x_ref[...], w_ref[...], preferred_element_type=jnp.float32)
        xs_ref[...] = xw * dinv[:N]

    return pl.pallas_call(
        body,
        out_shape=(
            jax.ShapeDtypeStruct((N, D), jnp.float32),
            jax.ShapeDtypeStruct((T, 1), jnp.float32),
        ),
    )(x, W0, degp)


def _bn(h, g, be):
    m = jnp.mean(h, axis=0, keepdims=True)
    v = jnp.mean((h - m) * (h - m), axis=0, keepdims=True)
    return (h - m) * lax.rsqrt(v + 1e-5) * g + be


def _tc_mid(aggp, xs0, dinv, b0, g0, be0, W1, N, D):
    def body(ap_ref, xs_ref, dv_ref, b_ref, g_ref, be_ref, w_ref, o_ref):
        agg = ap_ref[0, :N, :] + ap_ref[1, :N, :]
        dinv = dv_ref[:N]
        h = dinv * (agg + xs_ref[...]) + b_ref[...]
        y = jnp.maximum(_bn(h, g_ref[...], be_ref[...]), 0.0)
        o_ref[...] = (
            jnp.dot(y, w_ref[...], preferred_element_type=jnp.float32) * dinv
        )

    return pl.pallas_call(
        body,
        out_shape=jax.ShapeDtypeStruct((N, D), jnp.float32),
    )(aggp, xs0, dinv, b0, g0, be0, W1)


def _tc_final(aggp, xs1, dinv, b1, g1, be1, x, N, D):
    def body(ap_ref, xs_ref, dv_ref, b_ref, g_ref, be_ref, x_ref, o_ref):
        agg = ap_ref[0, :N, :] + ap_ref[1, :N, :]
        dinv = dv_ref[:N]
        h = dinv * (agg + xs_ref[...]) + b_ref[...]
        y = _bn(h, g_ref[...], be_ref[...])
        o_ref[...] = jnp.maximum(y + x_ref[...], 0.0)

    return pl.pallas_call(
        body,
        out_shape=jax.ShapeDtypeStruct((N, D), jnp.float32),
    )(aggp, xs1, dinv, b1, g1, be1, x)


def kernel(x, edge_index, W0, b0, g0, be0, W1, b1, g1, be1):
    N, D = x.shape
    E = edge_index.shape[1]
    NW = NC * NS
    AR = (-(-(N + 1) // (NS * CH))) * CH
    T = NS * AR

    src = edge_index[0]
    dst = edge_index[1]

    NCHD = -(-E // (NW * CH))
    EPD = NW * NCHD * CH
    dst_wd = jnp.concatenate(
        [dst, jnp.full((EPD - E,), N, jnp.int32)]).reshape(NW, NCHD, CH)

    NGT = -(-E // (NS * ACH * AB))
    NG0 = max(1, int(round(NGT * 0.25)))
    NG1 = NGT - NG0
    NGM = max(NG0, NG1)
    C0, C1 = NG0 * AB * ACH, NG1 * AB * ACH
    EPA = NS * (C0 + C1)

    def _part(a, fill):
        a = jnp.concatenate(
            [a, jnp.full((EPA - E,), fill, jnp.int32)])
        a0 = a[:NS * C0].reshape(NS, NG0, AB, ACH)
        a1 = a[NS * C0:].reshape(NS, NG1, AB, ACH)
        a0 = jnp.pad(a0, ((0, 0), (0, NGM - NG0), (0, 0), (0, 0)),
                     constant_values=fill)
        a1 = jnp.pad(a1, ((0, 0), (0, NGM - NG1), (0, 0), (0, 0)),
                     constant_values=fill)
        return jnp.concatenate([a0, a1], axis=0)

    idx_tab = jnp.concatenate([_part(src, 0), _part(dst, N)], axis=2)

    degp = _deg_partials(dst_wd, T, AR, NCHD, D)
    xs0, dinv = _tc_prep(x, W0, degp, N, T, D)
    aggp0 = _agg_partials(xs0, idx_tab, T, AR, NG0, NG1, D)
    xs1 = _tc_mid(aggp0, xs0, dinv, b0.reshape(1, D), g0.reshape(1, D),
                  be0.reshape(1, D), W1, N, D)
    aggp1 = _agg_partials(xs1, idx_tab, T, AR, NG0, NG1, D)
    return _tc_final(aggp1, xs1, dinv, b1.reshape(1, D), g1.reshape(1, D),
                     be1.reshape(1, D), x, N, D)


# speedup vs baseline: 1.0656x; 1.0656x over previous
import functools

import jax
import jax.numpy as jnp
from jax import lax
from jax.experimental import pallas as pl
from jax.experimental.pallas import tpu as pltpu
from jax.experimental.pallas import tpu_sc as plsc

NC = 2
NS = 16
CH = 128
W16 = 16


def _sc_mesh():
    return plsc.VectorSubcoreMesh(core_axis_name="c", subcore_axis_name="s")


def _deg_partials(dst_w, T, AR, NCH, D):
    @functools.partial(
        pl.kernel,
        out_type=jax.ShapeDtypeStruct((NC, T, D), jnp.float32),
        mesh=_sc_mesh(),
        scratch_types=[
            pltpu.VMEM((NCH, CH), jnp.int32),
            pltpu.VMEM((CH, D), jnp.float32),
            pltpu.VMEM_SHARED((T, D), jnp.float32),
            pltpu.SemaphoreType.DMA,
        ],
    )
    def deg_k(dstw_h, degp_h, idx_v, buf_v, dacc, sem):
        cid = lax.axis_index("c")
        sid = lax.axis_index("s")
        w = cid * NS + sid
        pltpu.sync_copy(dstw_h.at[w], idx_v)
        for i in range(CH):
            for c in range(D // 16):
                buf_v[i, pl.ds(c * 16, 16)] = jnp.zeros((16,), jnp.float32)
        base = sid * AR
        for c in range(AR // CH):
            pltpu.sync_copy(buf_v, dacc.at[pl.ds(base + c * CH, CH)])
        for i in range(CH):
            for c in range(D // 16):
                buf_v[i, pl.ds(c * 16, 16)] = jnp.ones((16,), jnp.float32)
        plsc.subcore_barrier()
        descs = [
            pltpu.async_copy(buf_v, dacc.at[idx_v.at[j]], sem, add=True)
            for j in range(NCH)
        ]
        for dsc in descs:
            dsc.wait()
        plsc.subcore_barrier()
        for c in range(AR // CH):
            pltpu.sync_copy(dacc.at[pl.ds(base + c * CH, CH)], buf_v)
            pltpu.sync_copy(buf_v, degp_h.at[cid, pl.ds(base + c * CH, CH)])

    return deg_k(dst_w)


AB = 2
ACH = 80


def _agg_partials(xs, idx_tab, T, AR, NG0, NG1, D):
    @functools.partial(
        pl.kernel,
        out_type=jax.ShapeDtypeStruct((NC, T, D), jnp.float32),
        mesh=_sc_mesh(),
        scratch_types=[
            pltpu.VMEM((2, 2 * AB, ACH), jnp.int32),
            pltpu.VMEM((AB, ACH, D), jnp.float32),
            pltpu.VMEM_SHARED((T, D), jnp.float32),
        ] + [pltpu.SemaphoreType.DMA] * (2 + 2 * AB),
    )
    def agg_k(xs_h, idx_h, aggp_h, idxb, rows, acc, *sems):
        isem, gsem, ssem = sems[:2], sems[2:2 + AB], sems[2 + AB:]
        cid = lax.axis_index("c")
        sid = lax.axis_index("s")
        w = cid * NS + sid
        for i in range(ACH):
            for c in range(D // 16):
                rows[0, i, pl.ds(c * 16, 16)] = jnp.zeros((16,), jnp.float32)
        base = sid * AR
        for c in range(AR // ACH):
            pltpu.sync_copy(rows.at[0], acc.at[pl.ds(base + c * ACH, ACH)])
        plsc.subcore_barrier()

        def run(NG):
            pltpu.sync_copy(idx_h.at[w, 0], idxb.at[0])
            idesc = [None, None]
            if NG > 1:
                idesc[1] = pltpu.async_copy(idx_h.at[w, 1], idxb.at[1], isem[1])
            gd = [
                pltpu.async_copy(xs_h.at[idxb.at[0, b]], rows.at[b], gsem[b])
                for b in range(AB)
            ]
            sd = [None] * AB
            for g in range(NG):
                p = g % 2
                for b in range(AB):
                    gd[b].wait()
                    sd[b] = pltpu.async_copy(
                        rows.at[b], acc.at[idxb.at[p, AB + b]], ssem[b],
                        add=True)
                if g + 1 < NG:
                    idesc[1 - p].wait()
                    for b in range(AB):
                        sd[b].wait()
                        gd[b] = pltpu.async_copy(
                            xs_h.at[idxb.at[1 - p, b]], rows.at[b], gsem[b])
                    if g + 2 < NG:
                        idesc[p] = pltpu.async_copy(
                            idx_h.at[w, g + 2], idxb.at[p], isem[p])
                else:
                    for b in range(AB):
                        sd[b].wait()

        @pl.when(cid == 0)
        def _():
            run(NG0)

        @pl.when(cid == 1)
        def _():
            run(NG1)

        plsc.subcore_barrier()
        for c in range(AR // ACH):
            pltpu.sync_copy(acc.at[pl.ds(base + c * ACH, ACH)], rows.at[0])
            pltpu.sync_copy(
                rows.at[0], aggp_h.at[cid, pl.ds(base + c * ACH, ACH)])

    return agg_k(xs, idx_tab)


def _tc_prep(x, W0, degp, N, T, D):
    def body(x_ref, w_ref, dp_ref, xs_ref, dinv_ref):
        deg = dp_ref[0, :, 0:1] + dp_ref[1, :, 0:1] + 1.0
        dinv = lax.rsqrt(deg)
        dinv_ref[...] = dinv
        xw = jnp.dot(x_ref[...], w_ref[...], preferred_element_type=jnp.float32)
        xs_ref[...] = xw * dinv[:N]

    return pl.pallas_call(
        body,
        out_shape=(
            jax.ShapeDtypeStruct((N, D), jnp.float32),
            jax.ShapeDtypeStruct((T, 1), jnp.float32),
        ),
    )(x, W0, degp)


def _bn(h, g, be):
    m = jnp.mean(h, axis=0, keepdims=True)
    v = jnp.mean((h - m) * (h - m), axis=0, keepdims=True)
    return (h - m) * lax.rsqrt(v + 1e-5) * g + be


def _tc_mid(aggp, xs0, dinv, b0, g0, be0, W1, N, D):
    def body(ap_ref, xs_ref, dv_ref, b_ref, g_ref, be_ref, w_ref, o_ref):
        agg = ap_ref[0, :N, :] + ap_ref[1, :N, :]
        dinv = dv_ref[:N]
        h = dinv * (agg + xs_ref[...]) + b_ref[...]
        y = jnp.maximum(_bn(h, g_ref[...], be_ref[...]), 0.0)
        o_ref[...] = (
            jnp.dot(y, w_ref[...], preferred_element_type=jnp.float32) * dinv
        )

    return pl.pallas_call(
        body,
        out_shape=jax.ShapeDtypeStruct((N, D), jnp.float32),
    )(aggp, xs0, dinv, b0, g0, be0, W1)


def _tc_final(aggp, xs1, dinv, b1, g1, be1, x, N, D):
    def body(ap_ref, xs_ref, dv_ref, b_ref, g_ref, be_ref, x_ref, o_ref):
        agg = ap_ref[0, :N, :] + ap_ref[1, :N, :]
        dinv = dv_ref[:N]
        h = dinv * (agg + xs_ref[...]) + b_ref[...]
        y = _bn(h, g_ref[...], be_ref[...])
        o_ref[...] = jnp.maximum(y + x_ref[...], 0.0)

    return pl.pallas_call(
        body,
        out_shape=jax.ShapeDtypeStruct((N, D), jnp.float32),
    )(aggp, xs1, dinv, b1, g1, be1, x)


def kernel(x, edge_index, W0, b0, g0, be0, W1, b1, g1, be1):
    N, D = x.shape
    E = edge_index.shape[1]
    NW = NC * NS
    AR = (-(-(N + 1) // (NS * CH))) * CH
    T = NS * AR

    src = edge_index[0]
    dst = edge_index[1]

    NCHD = -(-E // (NW * CH))
    EPD = NW * NCHD * CH
    dst_wd = jnp.concatenate(
        [dst, jnp.full((EPD - E,), N, jnp.int32)]).reshape(NW, NCHD, CH)

    NGT = -(-E // (NS * ACH * AB))
    NG0 = max(1, int(round(NGT * 0.31)))
    NG1 = NGT - NG0
    NGM = max(NG0, NG1)
    C0, C1 = NG0 * AB * ACH, NG1 * AB * ACH
    EPA = NS * (C0 + C1)

    def _part(a, fill):
        a = jnp.concatenate(
            [a, jnp.full((EPA - E,), fill, jnp.int32)])
        a0 = a[:NS * C0].reshape(NS, NG0, AB, ACH)
        a1 = a[NS * C0:].reshape(NS, NG1, AB, ACH)
        a0 = jnp.pad(a0, ((0, 0), (0, NGM - NG0), (0, 0), (0, 0)),
                     constant_values=fill)
        a1 = jnp.pad(a1, ((0, 0), (0, NGM - NG1), (0, 0), (0, 0)),
                     constant_values=fill)
        return jnp.concatenate([a0, a1], axis=0)

    idx_tab = jnp.concatenate([_part(src, 0), _part(dst, N)], axis=2)

    degp = _deg_partials(dst_wd, T, AR, NCHD, D)
    xs0, dinv = _tc_prep(x, W0, degp, N, T, D)
    aggp0 = _agg_partials(xs0, idx_tab, T, AR, NG0, NG1, D)
    xs1 = _tc_mid(aggp0, xs0, dinv, b0.reshape(1, D), g0.reshape(1, D),
                  be0.reshape(1, D), W1, N, D)
    aggp1 = _agg_partials(xs1, idx_tab, T, AR, NG0, NG1, D)
    return _tc_final(aggp1, xs1, dinv, b1.reshape(1, D), g1.reshape(1, D),
                     be1.reshape(1, D), x, N, D)


# speedup vs baseline: 1.1607x; 1.0892x over previous
import functools

import jax
import jax.numpy as jnp
from jax import lax
from jax.experimental import pallas as pl
from jax.experimental.pallas import tpu as pltpu
from jax.experimental.pallas import tpu_sc as plsc

NC = 2
NS = 16
CH = 128
W16 = 16


def _sc_mesh():
    return plsc.VectorSubcoreMesh(core_axis_name="c", subcore_axis_name="s")


def _deg_partials(dst_w, T, AR, NCH, D):
    @functools.partial(
        pl.kernel,
        out_type=jax.ShapeDtypeStruct((NC, T, D), jnp.float32),
        mesh=_sc_mesh(),
        scratch_types=[
            pltpu.VMEM((NCH, CH), jnp.int32),
            pltpu.VMEM((CH, D), jnp.float32),
            pltpu.VMEM_SHARED((T, D), jnp.float32),
            pltpu.SemaphoreType.DMA,
        ],
    )
    def deg_k(dstw_h, degp_h, idx_v, buf_v, dacc, sem):
        cid = lax.axis_index("c")
        sid = lax.axis_index("s")
        w = cid * NS + sid
        pltpu.sync_copy(dstw_h.at[w], idx_v)
        for i in range(CH):
            for c in range(D // 16):
                buf_v[i, pl.ds(c * 16, 16)] = jnp.zeros((16,), jnp.float32)
        base = sid * AR
        for c in range(AR // CH):
            pltpu.sync_copy(buf_v, dacc.at[pl.ds(base + c * CH, CH)])
        for i in range(CH):
            for c in range(D // 16):
                buf_v[i, pl.ds(c * 16, 16)] = jnp.ones((16,), jnp.float32)
        plsc.subcore_barrier()
        descs = [
            pltpu.async_copy(buf_v, dacc.at[idx_v.at[j]], sem, add=True)
            for j in range(NCH)
        ]
        for dsc in descs:
            dsc.wait()
        plsc.subcore_barrier()
        for c in range(AR // CH):
            pltpu.sync_copy(dacc.at[pl.ds(base + c * CH, CH)], buf_v)
            pltpu.sync_copy(buf_v, degp_h.at[cid, pl.ds(base + c * CH, CH)])

    return deg_k(dst_w)


AB = 2
ACH = 80


def _agg_partials(xs, idx_tab, T, AR, NG0, NG1, D):
    @functools.partial(
        pl.kernel,
        out_type=jax.ShapeDtypeStruct((NC, T, D), jnp.float32),
        mesh=_sc_mesh(),
        scratch_types=[
            pltpu.VMEM((2, 2 * AB, ACH), jnp.int32),
            pltpu.VMEM((AB, ACH, D), jnp.float32),
            pltpu.VMEM_SHARED((T, D), jnp.float32),
        ] + [pltpu.SemaphoreType.DMA] * (2 + 2 * AB),
    )
    def agg_k(xs_h, idx_h, aggp_h, idxb, rows, acc, *sems):
        isem, gsem, ssem = sems[:2], sems[2:2 + AB], sems[2 + AB:]
        cid = lax.axis_index("c")
        sid = lax.axis_index("s")
        w = cid * NS + sid
        for i in range(ACH):
            for c in range(D // 16):
                rows[0, i, pl.ds(c * 16, 16)] = jnp.zeros((16,), jnp.float32)
        base = sid * AR
        for c in range(AR // ACH):
            pltpu.sync_copy(rows.at[0], acc.at[pl.ds(base + c * ACH, ACH)])
        plsc.subcore_barrier()

        def run(NG):
            pltpu.sync_copy(idx_h.at[w, 0], idxb.at[0])
            idesc = [None, None]
            if NG > 1:
                idesc[1] = pltpu.async_copy(idx_h.at[w, 1], idxb.at[1], isem[1])
            gd = [
                pltpu.async_copy(xs_h.at[idxb.at[0, b]], rows.at[b], gsem[b])
                for b in range(AB)
            ]
            sd = [None] * AB
            for g in range(NG):
                p = g % 2
                for b in range(AB):
                    gd[b].wait()
                    sd[b] = pltpu.async_copy(
                        rows.at[b], acc.at[idxb.at[p, AB + b]], ssem[b],
                        add=True)
                if g + 1 < NG:
                    idesc[1 - p].wait()
                    for b in range(AB):
                        sd[b].wait()
                        gd[b] = pltpu.async_copy(
                            xs_h.at[idxb.at[1 - p, b]], rows.at[b], gsem[b])
                    if g + 2 < NG:
                        idesc[p] = pltpu.async_copy(
                            idx_h.at[w, g + 2], idxb.at[p], isem[p])
                else:
                    for b in range(AB):
                        sd[b].wait()

        @pl.when(cid == 0)
        def _():
            run(NG0)

        @pl.when(cid == 1)
        def _():
            run(NG1)

        plsc.subcore_barrier()
        for c in range(AR // ACH):
            pltpu.sync_copy(acc.at[pl.ds(base + c * ACH, ACH)], rows.at[0])
            pltpu.sync_copy(
                rows.at[0], aggp_h.at[cid, pl.ds(base + c * ACH, ACH)])

    return agg_k(xs, idx_tab)


def _tc_prep(x, W0, degp, N, T, D):
    def body(x_ref, w_ref, dp_ref, xs_ref, dinv_ref):
        deg = dp_ref[0, :, 0:1] + dp_ref[1, :, 0:1] + 1.0
        dinv = lax.rsqrt(deg)
        dinv_ref[...] = dinv
        xw = jnp.dot(x_ref[...], w_ref[...], preferred_element_type=jnp.float32)
        xs_ref[...] = xw * dinv[:N]

    return pl.pallas_call(
        body,
        out_shape=(
            jax.ShapeDtypeStruct((N, D), jnp.float32),
            jax.ShapeDtypeStruct((T, 1), jnp.float32),
        ),
    )(x, W0, degp)


def _bn(h, g, be):
    m = jnp.mean(h, axis=0, keepdims=True)
    v = jnp.mean((h - m) * (h - m), axis=0, keepdims=True)
    return (h - m) * lax.rsqrt(v + 1e-5) * g + be


def _tc_mid(aggp, xs0, dinv, b0, g0, be0, W1, N, D):
    def body(ap_ref, xs_ref, dv_ref, b_ref, g_ref, be_ref, w_ref, o_ref):
        agg = ap_ref[0, :N, :] + ap_ref[1, :N, :]
        dinv = dv_ref[:N]
        h = dinv * (agg + xs_ref[...]) + b_ref[...]
        y = jnp.maximum(_bn(h, g_ref[...], be_ref[...]), 0.0)
        o_ref[...] = (
            jnp.dot(y, w_ref[...], preferred_element_type=jnp.float32) * dinv
        )

    return pl.pallas_call(
        body,
        out_shape=jax.ShapeDtypeStruct((N, D), jnp.float32),
    )(aggp, xs0, dinv, b0, g0, be0, W1)


def _tc_final(aggp, xs1, dinv, b1, g1, be1, x, N, D):
    def body(ap_ref, xs_ref, dv_ref, b_ref, g_ref, be_ref, x_ref, o_ref):
        agg = ap_ref[0, :N, :] + ap_ref[1, :N, :]
        dinv = dv_ref[:N]
        h = dinv * (agg + xs_ref[...]) + b_ref[...]
        y = _bn(h, g_ref[...], be_ref[...])
        o_ref[...] = jnp.maximum(y + x_ref[...], 0.0)

    return pl.pallas_call(
        body,
        out_shape=jax.ShapeDtypeStruct((N, D), jnp.float32),
    )(aggp, xs1, dinv, b1, g1, be1, x)


def kernel(x, edge_index, W0, b0, g0, be0, W1, b1, g1, be1):
    N, D = x.shape
    E = edge_index.shape[1]
    NW = NC * NS
    AR = (-(-(N + 1) // (NS * CH))) * CH
    T = NS * AR

    src = edge_index[0]
    dst = edge_index[1]

    NCHD = -(-E // (NW * CH))
    EPD = NW * NCHD * CH
    dst_wd = jnp.concatenate(
        [dst, jnp.full((EPD - E,), N, jnp.int32)]).reshape(NW, NCHD, CH)

    NGT = -(-E // (NS * ACH * AB))
    NG0 = max(1, int(round(NGT * 0.40)))
    NG1 = NGT - NG0
    NGM = max(NG0, NG1)
    C0, C1 = NG0 * AB * ACH, NG1 * AB * ACH
    EPA = NS * (C0 + C1)

    def _part(a, fill):
        a = jnp.concatenate(
            [a, jnp.full((EPA - E,), fill, jnp.int32)])
        a0 = a[:NS * C0].reshape(NS, NG0, AB, ACH)
        a1 = a[NS * C0:].reshape(NS, NG1, AB, ACH)
        a0 = jnp.pad(a0, ((0, 0), (0, NGM - NG0), (0, 0), (0, 0)),
                     constant_values=fill)
        a1 = jnp.pad(a1, ((0, 0), (0, NGM - NG1), (0, 0), (0, 0)),
                     constant_values=fill)
        return jnp.concatenate([a0, a1], axis=0)

    idx_tab = jnp.concatenate([_part(src, 0), _part(dst, N)], axis=2)

    degp = _deg_partials(dst_wd, T, AR, NCHD, D)
    xs0, dinv = _tc_prep(x, W0, degp, N, T, D)
    aggp0 = _agg_partials(xs0, idx_tab, T, AR, NG0, NG1, D)
    xs1 = _tc_mid(aggp0, xs0, dinv, b0.reshape(1, D), g0.reshape(1, D),
                  be0.reshape(1, D), W1, N, D)
    aggp1 = _agg_partials(xs1, idx_tab, T, AR, NG0, NG1, D)
    return _tc_final(aggp1, xs1, dinv, b1.reshape(1, D), g1.reshape(1, D),
                     be1.reshape(1, D), x, N, D)


# speedup vs baseline: 1.2176x; 1.0491x over previous
import functools

import jax
import jax.numpy as jnp
from jax import lax
from jax.experimental import pallas as pl
from jax.experimental.pallas import tpu as pltpu
from jax.experimental.pallas import tpu_sc as plsc

NC = 2
NS = 16
CH = 128
W16 = 16


def _sc_mesh():
    return plsc.VectorSubcoreMesh(core_axis_name="c", subcore_axis_name="s")


def _deg_partials(dst_w, T, AR, NCH, D):
    @functools.partial(
        pl.kernel,
        out_type=jax.ShapeDtypeStruct((NC, T, D), jnp.float32),
        mesh=_sc_mesh(),
        scratch_types=[
            pltpu.VMEM((NCH, CH), jnp.int32),
            pltpu.VMEM((CH, D), jnp.float32),
            pltpu.VMEM_SHARED((T, D), jnp.float32),
            pltpu.SemaphoreType.DMA,
        ],
    )
    def deg_k(dstw_h, degp_h, idx_v, buf_v, dacc, sem):
        cid = lax.axis_index("c")
        sid = lax.axis_index("s")
        w = cid * NS + sid
        pltpu.sync_copy(dstw_h.at[w], idx_v)
        for i in range(CH):
            for c in range(D // 16):
                buf_v[i, pl.ds(c * 16, 16)] = jnp.zeros((16,), jnp.float32)
        base = sid * AR
        for c in range(AR // CH):
            pltpu.sync_copy(buf_v, dacc.at[pl.ds(base + c * CH, CH)])
        for i in range(CH):
            for c in range(D // 16):
                buf_v[i, pl.ds(c * 16, 16)] = jnp.ones((16,), jnp.float32)
        plsc.subcore_barrier()
        descs = [
            pltpu.async_copy(buf_v, dacc.at[idx_v.at[j]], sem, add=True)
            for j in range(NCH)
        ]
        for dsc in descs:
            dsc.wait()
        plsc.subcore_barrier()
        for c in range(AR // CH):
            pltpu.sync_copy(dacc.at[pl.ds(base + c * CH, CH)], buf_v)
            pltpu.sync_copy(buf_v, degp_h.at[cid, pl.ds(base + c * CH, CH)])

    return deg_k(dst_w)


AB = 2
ACH = 80


def _agg_partials(xs, idx_tab, T, AR, NG0, NG1, D):
    @functools.partial(
        pl.kernel,
        out_type=jax.ShapeDtypeStruct((NC, T, D), jnp.float32),
        mesh=_sc_mesh(),
        scratch_types=[
            pltpu.VMEM((2, 2 * AB, ACH), jnp.int32),
            pltpu.VMEM((AB, ACH, D), jnp.float32),
            pltpu.VMEM_SHARED((T, D), jnp.float32),
        ] + [pltpu.SemaphoreType.DMA] * (2 + 2 * AB),
    )
    def agg_k(xs_h, idx_h, aggp_h, idxb, rows, acc, *sems):
        isem, gsem, ssem = sems[:2], sems[2:2 + AB], sems[2 + AB:]
        cid = lax.axis_index("c")
        sid = lax.axis_index("s")
        w = cid * NS + sid
        for i in range(ACH):
            for c in range(D // 16):
                rows[0, i, pl.ds(c * 16, 16)] = jnp.zeros((16,), jnp.float32)
        base = sid * AR
        for c in range(AR // ACH):
            pltpu.sync_copy(rows.at[0], acc.at[pl.ds(base + c * ACH, ACH)])
        plsc.subcore_barrier()

        def run(NG):
            pltpu.sync_copy(idx_h.at[w, 0], idxb.at[0])
            idesc = [None, None]
            if NG > 1:
                idesc[1] = pltpu.async_copy(idx_h.at[w, 1], idxb.at[1], isem[1])
            gd = [
                pltpu.async_copy(xs_h.at[idxb.at[0, b]], rows.at[b], gsem[b])
                for b in range(AB)
            ]
            sd = [None] * AB
            for g in range(NG):
                p = g % 2
                for b in range(AB):
                    gd[b].wait()
                    sd[b] = pltpu.async_copy(
                        rows.at[b], acc.at[idxb.at[p, AB + b]], ssem[b],
                        add=True)
                if g + 1 < NG:
                    idesc[1 - p].wait()
                    for b in range(AB):
                        sd[b].wait()
                        gd[b] = pltpu.async_copy(
                            xs_h.at[idxb.at[1 - p, b]], rows.at[b], gsem[b])
                    if g + 2 < NG:
                        idesc[p] = pltpu.async_copy(
                            idx_h.at[w, g + 2], idxb.at[p], isem[p])
                else:
                    for b in range(AB):
                        sd[b].wait()

        @pl.when(cid == 0)
        def _():
            run(NG0)

        @pl.when(cid == 1)
        def _():
            run(NG1)

        plsc.subcore_barrier()
        for c in range(AR // ACH):
            pltpu.sync_copy(acc.at[pl.ds(base + c * ACH, ACH)], rows.at[0])
            pltpu.sync_copy(
                rows.at[0], aggp_h.at[cid, pl.ds(base + c * ACH, ACH)])

    return agg_k(xs, idx_tab)


def _tc_prep(x, W0, degp, N, T, D):
    def body(x_ref, w_ref, dp_ref, xs_ref, dinv_ref):
        deg = dp_ref[0, :, 0:1] + dp_ref[1, :, 0:1] + 1.0
        dinv = lax.rsqrt(deg)
        dinv_ref[...] = dinv
        xw = jnp.dot(x_ref[...], w_ref[...], preferred_element_type=jnp.float32)
        xs_ref[...] = xw * dinv[:N]

    return pl.pallas_call(
        body,
        out_shape=(
            jax.ShapeDtypeStruct((N, D), jnp.float32),
            jax.ShapeDtypeStruct((T, 1), jnp.float32),
        ),
    )(x, W0, degp)


def _bn(h, g, be):
    m = jnp.mean(h, axis=0, keepdims=True)
    v = jnp.mean((h - m) * (h - m), axis=0, keepdims=True)
    return (h - m) * lax.rsqrt(v + 1e-5) * g + be


def _tc_mid(aggp, xs0, dinv, b0, g0, be0, W1, N, D):
    def body(ap_ref, xs_ref, dv_ref, b_ref, g_ref, be_ref, w_ref, o_ref):
        agg = ap_ref[0, :N, :] + ap_ref[1, :N, :]
        dinv = dv_ref[:N]
        h = dinv * (agg + xs_ref[...]) + b_ref[...]
        y = jnp.maximum(_bn(h, g_ref[...], be_ref[...]), 0.0)
        o_ref[...] = (
            jnp.dot(y, w_ref[...], preferred_element_type=jnp.float32) * dinv
        )

    return pl.pallas_call(
        body,
        out_shape=jax.ShapeDtypeStruct((N, D), jnp.float32),
    )(aggp, xs0, dinv, b0, g0, be0, W1)


def _tc_final(aggp, xs1, dinv, b1, g1, be1, x, N, D):
    def body(ap_ref, xs_ref, dv_ref, b_ref, g_ref, be_ref, x_ref, o_ref):
        agg = ap_ref[0, :N, :] + ap_ref[1, :N, :]
        dinv = dv_ref[:N]
        h = dinv * (agg + xs_ref[...]) + b_ref[...]
        y = _bn(h, g_ref[...], be_ref[...])
        o_ref[...] = jnp.maximum(y + x_ref[...], 0.0)

    return pl.pallas_call(
        body,
        out_shape=jax.ShapeDtypeStruct((N, D), jnp.float32),
    )(aggp, xs1, dinv, b1, g1, be1, x)


def kernel(x, edge_index, W0, b0, g0, be0, W1, b1, g1, be1):
    N, D = x.shape
    E = edge_index.shape[1]
    NW = NC * NS
    AR = (-(-(N + 1) // (NS * CH))) * CH
    T = NS * AR

    src = edge_index[0]
    dst = edge_index[1]

    NCHD = -(-E // (NW * CH))
    EPD = NW * NCHD * CH
    dst_wd = jnp.concatenate(
        [dst, jnp.full((EPD - E,), N, jnp.int32)]).reshape(NW, NCHD, CH)

    NGT = -(-E // (NS * ACH * AB))
    NG0 = max(1, int(round(NGT * 0.44)))
    NG1 = NGT - NG0
    NGM = max(NG0, NG1)
    C0, C1 = NG0 * AB * ACH, NG1 * AB * ACH
    EPA = NS * (C0 + C1)

    def _part(a, fill):
        a = jnp.concatenate(
            [a, jnp.full((EPA - E,), fill, jnp.int32)])
        a0 = a[:NS * C0].reshape(NS, NG0, AB, ACH)
        a1 = a[NS * C0:].reshape(NS, NG1, AB, ACH)
        a0 = jnp.pad(a0, ((0, 0), (0, NGM - NG0), (0, 0), (0, 0)),
                     constant_values=fill)
        a1 = jnp.pad(a1, ((0, 0), (0, NGM - NG1), (0, 0), (0, 0)),
                     constant_values=fill)
        return jnp.concatenate([a0, a1], axis=0)

    idx_tab = jnp.concatenate([_part(src, 0), _part(dst, N)], axis=2)

    degp = _deg_partials(dst_wd, T, AR, NCHD, D)
    xs0, dinv = _tc_prep(x, W0, degp, N, T, D)
    aggp0 = _agg_partials(xs0, idx_tab, T, AR, NG0, NG1, D)
    xs1 = _tc_mid(aggp0, xs0, dinv, b0.reshape(1, D), g0.reshape(1, D),
                  be0.reshape(1, D), W1, N, D)
    aggp1 = _agg_partials(xs1, idx_tab, T, AR, NG0, NG1, D)
    return _tc_final(aggp1, xs1, dinv, b1.reshape(1, D), g1.reshape(1, D),
                     be1.reshape(1, D), x, N, D)


# speedup vs baseline: 1.2725x; 1.0451x over previous
import functools

import jax
import jax.numpy as jnp
from jax import lax
from jax.experimental import pallas as pl
from jax.experimental.pallas import tpu as pltpu
from jax.experimental.pallas import tpu_sc as plsc

NC = 2
NS = 16
CH = 128
W16 = 16


def _sc_mesh():
    return plsc.VectorSubcoreMesh(core_axis_name="c", subcore_axis_name="s")


def _deg_partials(dst_w, T, AR, NCH, D):
    @functools.partial(
        pl.kernel,
        out_type=jax.ShapeDtypeStruct((NC, T, D), jnp.float32),
        mesh=_sc_mesh(),
        scratch_types=[
            pltpu.VMEM((NCH, CH), jnp.int32),
            pltpu.VMEM((CH, D), jnp.float32),
            pltpu.VMEM_SHARED((T, D), jnp.float32),
            pltpu.SemaphoreType.DMA,
        ],
    )
    def deg_k(dstw_h, degp_h, idx_v, buf_v, dacc, sem):
        cid = lax.axis_index("c")
        sid = lax.axis_index("s")
        w = cid * NS + sid
        pltpu.sync_copy(dstw_h.at[w], idx_v)
        for i in range(CH):
            for c in range(D // 16):
                buf_v[i, pl.ds(c * 16, 16)] = jnp.zeros((16,), jnp.float32)
        base = sid * AR
        for c in range(AR // CH):
            pltpu.sync_copy(buf_v, dacc.at[pl.ds(base + c * CH, CH)])
        for i in range(CH):
            for c in range(D // 16):
                buf_v[i, pl.ds(c * 16, 16)] = jnp.ones((16,), jnp.float32)
        plsc.subcore_barrier()
        descs = [
            pltpu.async_copy(buf_v, dacc.at[idx_v.at[j]], sem, add=True)
            for j in range(NCH)
        ]
        for dsc in descs:
            dsc.wait()
        plsc.subcore_barrier()
        for c in range(AR // CH):
            pltpu.sync_copy(dacc.at[pl.ds(base + c * CH, CH)], buf_v)
            pltpu.sync_copy(buf_v, degp_h.at[cid, pl.ds(base + c * CH, CH)])

    return deg_k(dst_w)


AB = 2
ACH = 80


def _agg_partials(xs, idx_tab, T, AR, NG0, NG1, D):
    @functools.partial(
        pl.kernel,
        out_type=jax.ShapeDtypeStruct((NC, T, D), jnp.float32),
        mesh=_sc_mesh(),
        scratch_types=[
            pltpu.VMEM((2, 2 * AB, ACH), jnp.int32),
            pltpu.VMEM((AB, ACH, D), jnp.float32),
            pltpu.VMEM_SHARED((T, D), jnp.float32),
        ] + [pltpu.SemaphoreType.DMA] * (2 + 2 * AB),
    )
    def agg_k(xs_h, idx_h, aggp_h, idxb, rows, acc, *sems):
        isem, gsem, ssem = sems[:2], sems[2:2 + AB], sems[2 + AB:]
        cid = lax.axis_index("c")
        sid = lax.axis_index("s")
        w = cid * NS + sid
        for i in range(ACH):
            for c in range(D // 16):
                rows[0, i, pl.ds(c * 16, 16)] = jnp.zeros((16,), jnp.float32)
        base = sid * AR
        for c in range(AR // ACH):
            pltpu.sync_copy(rows.at[0], acc.at[pl.ds(base + c * ACH, ACH)])
        plsc.subcore_barrier()

        def run(NG):
            pltpu.sync_copy(idx_h.at[w, 0], idxb.at[0])
            idesc = [None, None]
            if NG > 1:
                idesc[1] = pltpu.async_copy(idx_h.at[w, 1], idxb.at[1], isem[1])
            gd = [
                pltpu.async_copy(xs_h.at[idxb.at[0, b]], rows.at[b], gsem[b])
                for b in range(AB)
            ]
            sd = [None] * AB
            for g in range(NG):
                p = g % 2
                for b in range(AB):
                    gd[b].wait()
                    sd[b] = pltpu.async_copy(
                        rows.at[b], acc.at[idxb.at[p, AB + b]], ssem[b],
                        add=True)
                if g + 1 < NG:
                    idesc[1 - p].wait()
                    for b in range(AB):
                        sd[b].wait()
                        gd[b] = pltpu.async_copy(
                            xs_h.at[idxb.at[1 - p, b]], rows.at[b], gsem[b])
                    if g + 2 < NG:
                        idesc[p] = pltpu.async_copy(
                            idx_h.at[w, g + 2], idxb.at[p], isem[p])
                else:
                    for b in range(AB):
                        sd[b].wait()

        @pl.when(cid == 0)
        def _():
            run(NG0)

        @pl.when(cid == 1)
        def _():
            run(NG1)

        plsc.subcore_barrier()
        for c in range(AR // ACH):
            pltpu.sync_copy(acc.at[pl.ds(base + c * ACH, ACH)], rows.at[0])
            pltpu.sync_copy(
                rows.at[0], aggp_h.at[cid, pl.ds(base + c * ACH, ACH)])

    return agg_k(xs, idx_tab)


def _tc_prep(x, W0, degp, N, T, D):
    def body(x_ref, w_ref, dp_ref, xs_ref, dinv_ref):
        deg = dp_ref[0, :, 0:1] + dp_ref[1, :, 0:1] + 1.0
        dinv = lax.rsqrt(deg)
        dinv_ref[...] = dinv
        xw = jnp.dot(x_ref[...], w_ref[...], preferred_element_type=jnp.float32)
        xs_ref[...] = xw * dinv[:N]

    return pl.pallas_call(
        body,
        out_shape=(
            jax.ShapeDtypeStruct((N, D), jnp.float32),
            jax.ShapeDtypeStruct((T, 1), jnp.float32),
        ),
    )(x, W0, degp)


def _bn(h, g, be):
    m = jnp.mean(h, axis=0, keepdims=True)
    v = jnp.mean((h - m) * (h - m), axis=0, keepdims=True)
    return (h - m) * lax.rsqrt(v + 1e-5) * g + be


def _tc_mid(aggp, xs0, dinv, b0, g0, be0, W1, N, D):
    def body(ap_ref, xs_ref, dv_ref, b_ref, g_ref, be_ref, w_ref, o_ref):
        agg = ap_ref[0, :N, :] + ap_ref[1, :N, :]
        dinv = dv_ref[:N]
        h = dinv * (agg + xs_ref[...]) + b_ref[...]
        y = jnp.maximum(_bn(h, g_ref[...], be_ref[...]), 0.0)
        o_ref[...] = (
            jnp.dot(y, w_ref[...], preferred_element_type=jnp.float32) * dinv
        )

    return pl.pallas_call(
        body,
        out_shape=jax.ShapeDtypeStruct((N, D), jnp.float32),
    )(aggp, xs0, dinv, b0, g0, be0, W1)


def _tc_final(aggp, xs1, dinv, b1, g1, be1, x, N, D):
    def body(ap_ref, xs_ref, dv_ref, b_ref, g_ref, be_ref, x_ref, o_ref):
        agg = ap_ref[0, :N, :] + ap_ref[1, :N, :]
        dinv = dv_ref[:N]
        h = dinv * (agg + xs_ref[...]) + b_ref[...]
        y = _bn(h, g_ref[...], be_ref[...])
        o_ref[...] = jnp.maximum(y + x_ref[...], 0.0)

    return pl.pallas_call(
        body,
        out_shape=jax.ShapeDtypeStruct((N, D), jnp.float32),
    )(aggp, xs1, dinv, b1, g1, be1, x)


def kernel(x, edge_index, W0, b0, g0, be0, W1, b1, g1, be1):
    N, D = x.shape
    E = edge_index.shape[1]
    NW = NC * NS
    AR = (-(-(N + 1) // (NS * CH))) * CH
    T = NS * AR

    src = edge_index[0]
    dst = edge_index[1]

    NCHD = -(-E // (NW * CH))
    EPD = NW * NCHD * CH
    dst_wd = jnp.concatenate(
        [dst, jnp.full((EPD - E,), N, jnp.int32)]).reshape(NW, NCHD, CH)

    NGT = -(-E // (NS * ACH * AB))
    NG0 = max(1, int(round(NGT * 0.48)))
    NG1 = NGT - NG0
    NGM = max(NG0, NG1)
    C0, C1 = NG0 * AB * ACH, NG1 * AB * ACH
    EPA = NS * (C0 + C1)

    def _part(a, fill):
        a = jnp.concatenate(
            [a, jnp.full((EPA - E,), fill, jnp.int32)])
        a0 = a[:NS * C0].reshape(NS, NG0, AB, ACH)
        a1 = a[NS * C0:].reshape(NS, NG1, AB, ACH)
        a0 = jnp.pad(a0, ((0, 0), (0, NGM - NG0), (0, 0), (0, 0)),
                     constant_values=fill)
        a1 = jnp.pad(a1, ((0, 0), (0, NGM - NG1), (0, 0), (0, 0)),
                     constant_values=fill)
        return jnp.concatenate([a0, a1], axis=0)

    idx_tab = jnp.concatenate([_part(src, 0), _part(dst, N)], axis=2)

    degp = _deg_partials(dst_wd, T, AR, NCHD, D)
    xs0, dinv = _tc_prep(x, W0, degp, N, T, D)
    aggp0 = _agg_partials(xs0, idx_tab, T, AR, NG0, NG1, D)
    xs1 = _tc_mid(aggp0, xs0, dinv, b0.reshape(1, D), g0.reshape(1, D),
                  be0.reshape(1, D), W1, N, D)
    aggp1 = _agg_partials(xs1, idx_tab, T, AR, NG0, NG1, D)
    return _tc_final(aggp1, xs1, dinv, b1.reshape(1, D), g1.reshape(1, D),
                     be1.reshape(1, D), x, N, D)


# speedup vs baseline: 1.2977x; 1.0198x over previous
import functools

import jax
import jax.numpy as jnp
from jax import lax
from jax.experimental import pallas as pl
from jax.experimental.pallas import tpu as pltpu
from jax.experimental.pallas import tpu_sc as plsc

NC = 2
NS = 16
CH = 128
W16 = 16


def _sc_mesh():
    return plsc.VectorSubcoreMesh(core_axis_name="c", subcore_axis_name="s")


def _deg_partials(dst_w, T, AR, NCH, D):
    @functools.partial(
        pl.kernel,
        out_type=jax.ShapeDtypeStruct((NC, T, D), jnp.float32),
        mesh=_sc_mesh(),
        scratch_types=[
            pltpu.VMEM((NCH, CH), jnp.int32),
            pltpu.VMEM((CH, D), jnp.float32),
            pltpu.VMEM_SHARED((T, D), jnp.float32),
            pltpu.SemaphoreType.DMA,
        ],
    )
    def deg_k(dstw_h, degp_h, idx_v, buf_v, dacc, sem):
        cid = lax.axis_index("c")
        sid = lax.axis_index("s")
        w = cid * NS + sid
        pltpu.sync_copy(dstw_h.at[w], idx_v)
        for i in range(CH):
            for c in range(D // 16):
                buf_v[i, pl.ds(c * 16, 16)] = jnp.zeros((16,), jnp.float32)
        base = sid * AR
        for c in range(AR // CH):
            pltpu.sync_copy(buf_v, dacc.at[pl.ds(base + c * CH, CH)])
        for i in range(CH):
            for c in range(D // 16):
                buf_v[i, pl.ds(c * 16, 16)] = jnp.ones((16,), jnp.float32)
        plsc.subcore_barrier()
        descs = [
            pltpu.async_copy(buf_v, dacc.at[idx_v.at[j]], sem, add=True)
            for j in range(NCH)
        ]
        for dsc in descs:
            dsc.wait()
        plsc.subcore_barrier()
        for c in range(AR // CH):
            pltpu.sync_copy(dacc.at[pl.ds(base + c * CH, CH)], buf_v)
            pltpu.sync_copy(buf_v, degp_h.at[cid, pl.ds(base + c * CH, CH)])

    return deg_k(dst_w)


AB = 2
ACH = 80


def _agg_partials(xs, idx_tab, T, AR, NG0, NG1, D):
    @functools.partial(
        pl.kernel,
        out_type=jax.ShapeDtypeStruct((NC, T, D), jnp.float32),
        mesh=_sc_mesh(),
        scratch_types=[
            pltpu.VMEM((2, 2 * AB, ACH), jnp.int32),
            pltpu.VMEM((AB, ACH, D), jnp.float32),
            pltpu.VMEM_SHARED((T, D), jnp.float32),
        ] + [pltpu.SemaphoreType.DMA] * (2 + 2 * AB),
    )
    def agg_k(xs_h, idx_h, aggp_h, idxb, rows, acc, *sems):
        isem, gsem, ssem = sems[:2], sems[2:2 + AB], sems[2 + AB:]
        cid = lax.axis_index("c")
        sid = lax.axis_index("s")
        w = cid * NS + sid
        for i in range(ACH):
            for c in range(D // 16):
                rows[0, i, pl.ds(c * 16, 16)] = jnp.zeros((16,), jnp.float32)
        base = sid * AR
        for c in range(AR // ACH):
            pltpu.sync_copy(rows.at[0], acc.at[pl.ds(base + c * ACH, ACH)])
        plsc.subcore_barrier()

        def run(NG):
            pltpu.sync_copy(idx_h.at[w, 0], idxb.at[0])
            idesc = [None, None]
            if NG > 1:
                idesc[1] = pltpu.async_copy(idx_h.at[w, 1], idxb.at[1], isem[1])
            gd = [
                pltpu.async_copy(xs_h.at[idxb.at[0, b]], rows.at[b], gsem[b])
                for b in range(AB)
            ]
            sd = [None] * AB
            for g in range(NG):
                p = g % 2
                for b in range(AB):
                    gd[b].wait()
                    sd[b] = pltpu.async_copy(
                        rows.at[b], acc.at[idxb.at[p, AB + b]], ssem[b],
                        add=True)
                if g + 1 < NG:
                    idesc[1 - p].wait()
                    for b in range(AB):
                        sd[b].wait()
                        gd[b] = pltpu.async_copy(
                            xs_h.at[idxb.at[1 - p, b]], rows.at[b], gsem[b])
                    if g + 2 < NG:
                        idesc[p] = pltpu.async_copy(
                            idx_h.at[w, g + 2], idxb.at[p], isem[p])
                else:
                    for b in range(AB):
                        sd[b].wait()

        @pl.when(cid == 0)
        def _():
            run(NG0)

        @pl.when(cid == 1)
        def _():
            run(NG1)

        plsc.subcore_barrier()
        for c in range(AR // ACH):
            pltpu.sync_copy(acc.at[pl.ds(base + c * ACH, ACH)], rows.at[0])
            pltpu.sync_copy(
                rows.at[0], aggp_h.at[cid, pl.ds(base + c * ACH, ACH)])

    return agg_k(xs, idx_tab)


def _tc_prep(x, W0, degp, N, T, D):
    def body(x_ref, w_ref, dp_ref, xs_ref, dinv_ref):
        deg = dp_ref[0, :, 0:1] + dp_ref[1, :, 0:1] + 1.0
        dinv = lax.rsqrt(deg)
        dinv_ref[...] = dinv
        xw = jnp.dot(x_ref[...], w_ref[...], preferred_element_type=jnp.float32)
        xs_ref[...] = xw * dinv[:N]

    return pl.pallas_call(
        body,
        out_shape=(
            jax.ShapeDtypeStruct((N, D), jnp.float32),
            jax.ShapeDtypeStruct((T, 1), jnp.float32),
        ),
    )(x, W0, degp)


def _bn(h, g, be):
    m = jnp.mean(h, axis=0, keepdims=True)
    v = jnp.mean((h - m) * (h - m), axis=0, keepdims=True)
    return (h - m) * lax.rsqrt(v + 1e-5) * g + be


def _tc_mid(aggp, xs0, dinv, b0, g0, be0, W1, N, D):
    def body(ap_ref, xs_ref, dv_ref, b_ref, g_ref, be_ref, w_ref, o_ref):
        agg = ap_ref[0, :N, :] + ap_ref[1, :N, :]
        dinv = dv_ref[:N]
        h = dinv * (agg + xs_ref[...]) + b_ref[...]
        y = jnp.maximum(_bn(h, g_ref[...], be_ref[...]), 0.0)
        o_ref[...] = (
            jnp.dot(y, w_ref[...], preferred_element_type=jnp.float32) * dinv
        )

    return pl.pallas_call(
        body,
        out_shape=jax.ShapeDtypeStruct((N, D), jnp.float32),
    )(aggp, xs0, dinv, b0, g0, be0, W1)


def _tc_final(aggp, xs1, dinv, b1, g1, be1, x, N, D):
    def body(ap_ref, xs_ref, dv_ref, b_ref, g_ref, be_ref, x_ref, o_ref):
        agg = ap_ref[0, :N, :] + ap_ref[1, :N, :]
        dinv = dv_ref[:N]
        h = dinv * (agg + xs_ref[...]) + b_ref[...]
        y = _bn(h, g_ref[...], be_ref[...])
        o_ref[...] = jnp.maximum(y + x_ref[...], 0.0)

    return pl.pallas_call(
        body,
        out_shape=jax.ShapeDtypeStruct((N, D), jnp.float32),
    )(aggp, xs1, dinv, b1, g1, be1, x)


def kernel(x, edge_index, W0, b0, g0, be0, W1, b1, g1, be1):
    N, D = x.shape
    E = edge_index.shape[1]
    NW = NC * NS
    AR = (-(-(N + 1) // (NS * CH))) * CH
    T = NS * AR

    src = edge_index[0]
    dst = edge_index[1]

    NCHD = -(-E // (NW * CH))
    EPD = NW * NCHD * CH
    dst_wd = jnp.concatenate(
        [dst, jnp.full((EPD - E,), N, jnp.int32)]).reshape(NW, NCHD, CH)

    NGT = -(-E // (NS * ACH * AB))
    NG0 = max(1, int(round(NGT * 0.50)))
    NG1 = NGT - NG0
    NGM = max(NG0, NG1)
    C0, C1 = NG0 * AB * ACH, NG1 * AB * ACH
    EPA = NS * (C0 + C1)

    def _part(a, fill):
        a = jnp.concatenate(
            [a, jnp.full((EPA - E,), fill, jnp.int32)])
        a0 = a[:NS * C0].reshape(NS, NG0, AB, ACH)
        a1 = a[NS * C0:].reshape(NS, NG1, AB, ACH)
        a0 = jnp.pad(a0, ((0, 0), (0, NGM - NG0), (0, 0), (0, 0)),
                     constant_values=fill)
        a1 = jnp.pad(a1, ((0, 0), (0, NGM - NG1), (0, 0), (0, 0)),
                     constant_values=fill)
        return jnp.concatenate([a0, a1], axis=0)

    idx_tab = jnp.concatenate([_part(src, 0), _part(dst, N)], axis=2)

    degp = _deg_partials(dst_wd, T, AR, NCHD, D)
    xs0, dinv = _tc_prep(x, W0, degp, N, T, D)
    aggp0 = _agg_partials(xs0, idx_tab, T, AR, NG0, NG1, D)
    xs1 = _tc_mid(aggp0, xs0, dinv, b0.reshape(1, D), g0.reshape(1, D),
                  be0.reshape(1, D), W1, N, D)
    aggp1 = _agg_partials(xs1, idx_tab, T, AR, NG0, NG1, D)
    return _tc_final(aggp1, xs1, dinv, b1.reshape(1, D), g1.reshape(1, D),
                     be1.reshape(1, D), x, N, D)
